# baseline (speedup 1.0000x reference)
import functools

import jax
import jax.numpy as jnp
from jax import lax
from jax.experimental import pallas as pl
from jax.experimental.pallas import tpu as pltpu
from jax.experimental.pallas import tpu_sc as plsc

_NC, _NS, _L = 2, 16, 16
_NW = _NC * _NS

_GATHER_DNUMS = lax.GatherDimensionNumbers(
    offset_dims=(), collapsed_slice_dims=(0,), start_index_map=(0,))


_TRASH = 144


def _lane_gather(v, idx):
    return lax.gather(v, idx[:, None], _GATHER_DNUMS, slice_sizes=(1,),
                      mode=lax.GatherScatterMode.PROMISE_IN_BOUNDS)


def _make_sc_kernel(N, D):
    rows_per_w = N // _NW
    groups = rows_per_w // _L
    mesh = plsc.VectorSubcoreMesh(core_axis_name="c", subcore_axis_name="s")

    @functools.partial(
        pl.kernel,
        out_type=jax.ShapeDtypeStruct((N, D), jnp.float32),
        mesh=mesh,
        scratch_types=[
            pltpu.VMEM((rows_per_w,), jnp.int32),
            pltpu.VMEM((rows_per_w + 2 * _L,), jnp.int32),
            pltpu.VMEM((rows_per_w + 2 * _L,), jnp.int32),
            pltpu.VMEM((_L, D), jnp.float32),
            pltpu.VMEM((_L, D), jnp.float32),
            pltpu.SemaphoreType.DMA,
            pltpu.SemaphoreType.DMA,
        ],
    )
    def sc_kernel(x_hbm, a_hbm, m_hbm, o_hbm, m_v, idxm, idxu, xbuf, abuf,
                  sem_in, sem_out):
        wid = lax.axis_index("s") * _NC + lax.axis_index("c")
        base = wid * rows_per_w

        pltpu.sync_copy(m_hbm.at[pl.ds(base, rows_per_w)], m_v)

        mc = jnp.int32(0)
        uc = jnp.int32(0)
        lane = lax.iota(jnp.int32, _L)
        padm = lane * 0
        padu = lane * 0
        for g in range(groups):
            mvec = m_v[pl.ds(g * _L, _L)]
            rowid = base + g * _L + lane
            cm = mvec
            for k in (1, 2, 4, 8):
                sh = _lane_gather(cm, jnp.maximum(lane - k, 0))
                cm = cm + jnp.where(lane >= k, sh, 0)
            nm = cm[_L - 1]
            pos = jnp.where(mvec != 0, cm - 1, nm + lane - cm)
            svals = lane
            for s in range(_L):
                svals = jnp.where(lane == pos[s], base + g * _L + s, svals)
            idxm[pl.ds(mc, _L)] = svals
            uvals = _lane_gather(svals, jnp.minimum(nm + lane, _L - 1))
            idxu[pl.ds(uc, _L)] = uvals
            dm = _lane_gather(svals, jnp.maximum(nm - 1, 0) + lane * 0)
            du = _lane_gather(uvals, jnp.maximum(_L - nm - 1, 0) + lane * 0)
            padm = jnp.where(lane * 0 + nm > 0, dm, padm)
            padu = jnp.where(lane * 0 + nm < _L, du, padu)
            mc = mc + nm
            uc = uc + (_L - nm)

        idxm[pl.ds(mc, _L)] = padm
        idxu[pl.ds(uc, _L)] = padu

        ngm = (mc + _L - 1) // _L
        ngu = (uc + _L - 1) // _L

        def masked_body(g, carry):
            idxvec = idxm[pl.ds(g * _L, _L)]
            cpx = pltpu.make_async_copy(x_hbm.at[idxvec], xbuf, sem_in)
            cpa = pltpu.make_async_copy(a_hbm.at[idxvec], abuf, sem_in)
            cpx.start()
            cpa.start()
            cpx.wait()
            cpa.wait()

            def add_body(j, c):
                w = pl.ds(j * _L, _L)
                for r in range(_L):
                    xbuf[r, w] = xbuf[r, w] + abuf[r, w]
                return c

            lax.fori_loop(0, D // _L, add_body, 0)
            cpo = pltpu.make_async_copy(xbuf, o_hbm.at[idxvec], sem_out)
            cpo.start()
            cpo.wait()
            return carry

        lax.fori_loop(0, ngm, masked_body, 0)

        def unmasked_body(g, carry):
            idxvec = idxu[pl.ds(g * _L, _L)]
            cpx = pltpu.make_async_copy(x_hbm.at[idxvec], xbuf, sem_in)
            cpx.start()
            cpx.wait()
            cpo = pltpu.make_async_copy(xbuf, o_hbm.at[idxvec], sem_out)
            cpo.start()
            cpo.wait()
            return carry

        lax.fori_loop(0, ngu, unmasked_body, 0)

    return sc_kernel


def kernel(x, attack, attack_mask):
    B, S, D = x.shape
    N = B * S
    x2 = x.reshape(N, D)
    a2 = attack.reshape(N, D)
    m2 = attack_mask.reshape(N).astype(jnp.int32)
    out = _make_sc_kernel(N, D)(x2, a2, m2)
    return out.reshape(B, S, D)
